# baseline (speedup 1.0000x reference)
import functools
import jax
import jax.numpy as jnp
from jax import lax
from jax.experimental import pallas as pl
from jax.experimental.pallas import tpu as pltpu
from jax.experimental.pallas import tpu_sc as plsc

NC = 2
NS = 16
NW = NC * NS
G = 64
CH = 16

_MASK_HI = -65536


def _ceil_to(a, b):
    return (a + b - 1) // b * b


def _lo(w):
    return plsc.bitcast(w << 16, jnp.float32)


def _hi(w):
    return plsc.bitcast(w & _MASK_HI, jnp.float32)


def _pack_tc(v):
    u = lax.bitcast_convert_type(v.astype(jnp.bfloat16), jnp.uint16)
    k = u.shape[-1] // 2
    return (u[:, :k].astype(jnp.int32) |
            (u[:, k:].astype(jnp.int32) << 16))


def _edense_body(ea_ref, w_ref, b_ref, e_ref):
    v = jnp.dot(ea_ref[...], w_ref[...], preferred_element_type=jnp.float32)
    v = v + b_ref[...]
    e_ref[...] = _pack_tc(v)


def _edense(ea_p, Wc, bc, eb):
    e_pad = ea_p.shape[0]
    grid = e_pad // eb
    return pl.pallas_call(
        _edense_body,
        grid=(grid,),
        in_specs=[
            pl.BlockSpec((eb, ea_p.shape[1]), lambda i: (i, 0)),
            pl.BlockSpec(Wc.shape, lambda i: (0, 0)),
            pl.BlockSpec(bc.shape, lambda i: (0, 0)),
        ],
        out_specs=pl.BlockSpec((eb, 64), lambda i: (i, 0)),
        out_shape=jax.ShapeDtypeStruct((e_pad, 64), jnp.int32),
    )(ea_p, Wc, bc)


def _xpack_body(x_ref, xw_ref):
    xw_ref[...] = _pack_tc(x_ref[...])


def _xpack(x, nb):
    n, d = x.shape
    return pl.pallas_call(
        _xpack_body,
        grid=(n // nb,),
        in_specs=[pl.BlockSpec((nb, d), lambda i: (i, 0))],
        out_specs=pl.BlockSpec((nb, d // 2), lambda i: (i, 0)),
        out_shape=jax.ShapeDtypeStruct((n, d // 2), jnp.int32),
    )(x)


def _make_agg(tw, ew, compute_rows, n_acc, ng):
    mesh = plsc.VectorSubcoreMesh(core_axis_name="c", subcore_axis_name="s")
    rows_per_tile = n_acc // NS
    nz = rows_per_tile // 128
    nsteps = ng
    assert nsteps % 2 == 0 and CH % 2 == 0 and nsteps % CH == 0

    @functools.partial(
        pl.kernel,
        mesh=mesh,
        out_type=jax.ShapeDtypeStruct((NC, n_acc, 128), jnp.float32),
        compiler_params=pltpu.CompilerParams(use_tc_tiling_on_sc=False,
                                             needs_layout_passes=False),
        scratch_types=[
            pltpu.VMEM((CH, G), jnp.int32),
            pltpu.VMEM((2, CH, G), jnp.int32),
            pltpu.VMEM((2, G, tw), jnp.int32),
            pltpu.VMEM((2, G, ew), jnp.int32),
            pltpu.VMEM((2, G, 128), jnp.float32),
            pltpu.SemaphoreType.DMA,
            pltpu.SemaphoreType.DMA,
            pltpu.SemaphoreType.DMA,
            pltpu.SemaphoreType.DMA,
            pltpu.SemaphoreType.DMA,
            pltpu.SemaphoreType.DMA,
            pltpu.VMEM_SHARED((n_acc, 128), jnp.float32),
        ],
    )
    def body(tbl_hbm, src_hbm, dst_hbm, emb_hbm, out_hbm,
             src_v, dst_v, tbl_v, emb_v, m_v, gsem0, gsem1, esem0, esem1,
             ssem0, ssem1, acc):
        c = lax.axis_index("c")
        s = lax.axis_index("s")
        w = s * NC + c
        gsem = (gsem0, gsem1)
        esem = (esem0, esem1)
        ssem = (ssem0, ssem1)

        zeros16 = jnp.zeros((16,), jnp.float32)

        def zb_body(r, carry):
            for cc in range(8):
                m_v[0, r, pl.ds(cc * 16, 16)] = zeros16
            return carry

        lax.fori_loop(0, G, zb_body, 0)
        tile_base = s * rows_per_tile

        def zacc_body(k, carry):
            for q in range(128 // G):
                pltpu.sync_copy(
                    m_v.at[0],
                    acc.at[pl.ds(tile_base + k * 128 + q * G, G)])
            return carry

        lax.fori_loop(0, nz, zacc_body, 0)
        plsc.subcore_barrier()

        def issue_loads(j, b):
            jj = lax.rem(j, CH)
            pltpu.async_copy(
                emb_hbm.at[pl.ds((w * ng + j) * G, G)], emb_v.at[b], esem[b])
            pltpu.async_copy(tbl_hbm.at[src_v.at[jj]], tbl_v.at[b], gsem[b])

        def wait_loads(b):
            pltpu.make_async_copy(
                emb_hbm.at[pl.ds(0, G)], emb_v.at[b], esem[b]).wait()
            pltpu.make_async_copy(
                tbl_hbm.at[src_v.at[0]], tbl_v.at[b], gsem[b]).wait()

        def issue_scatter(j, b):
            jj = lax.rem(j, CH)
            p = lax.rem(j // CH, 2)
            pltpu.async_copy(m_v.at[b], acc.at[dst_v.at[p, jj]], ssem[b],
                             add=True)

        def wait_scatter(b):
            pltpu.make_async_copy(
                m_v.at[b], acc.at[dst_v.at[0, 0]], ssem[b]).wait()

        def load_idx(blk):
            pltpu.sync_copy(src_hbm.at[pl.ds(w * ng + blk * CH, CH)], src_v)
            pltpu.sync_copy(dst_hbm.at[pl.ds(w * ng + blk * CH, CH)],
                            dst_v.at[lax.rem(blk, 2)])

        load_idx(0)
        issue_loads(0, 0)

        def step2(jj, carry):
            for b in range(2):
                j = jj * 2 + b
                wait_loads(b)
                if b == 0:
                    issue_loads(j + 1, 1)
                else:
                    nxt = j + 1

                    @pl.when(nxt < nsteps)
                    def _():
                        @pl.when(lax.rem(nxt, CH) == 0)
                        def _():
                            load_idx(nxt // CH)

                        issue_loads(nxt, 0)

                @pl.when(j >= 2)
                def _(b=b):
                    wait_scatter(b)

                def rows(r, rc, b=b):
                    compute_rows(tbl_v, emb_v, m_v, b, r)
                    return rc

                lax.fori_loop(0, G, rows, 0)
                issue_scatter(j, b)
            return carry

        lax.fori_loop(0, nsteps // 2, step2, 0)
        wait_scatter(0)
        wait_scatter(1)
        plsc.subcore_barrier()

        def wout(k, carry):
            r0 = tile_base + k * 128
            pltpu.sync_copy(acc.at[pl.ds(r0, 128)],
                            out_hbm.at[c, pl.ds(r0, 128)])
            return carry

        lax.fori_loop(0, nz, wout, 0)

    return body


def _rows_l1(tbl_v, emb_v, m_v, b, r):
    for g in range(4):
        wx = tbl_v[b, r, pl.ds(g * 16, 16)]
        we = emb_v[b, r, pl.ds(g * 16, 16)]
        m_v[b, r, pl.ds(g * 16, 16)] = jnp.maximum(_lo(wx) + _lo(we), 0.0)
        m_v[b, r, pl.ds(64 + g * 16, 16)] = jnp.maximum(_hi(wx) + _hi(we), 0.0)


def _rows_l23(tbl_v, emb_v, m_v, b, r):
    for g in range(2):
        wx = tbl_v[b, r, pl.ds(g * 16, 16)]
        xlo = _lo(wx)
        xhi = _hi(wx)
        we_a = emb_v[b, r, pl.ds(g * 16, 16)]
        we_b = emb_v[b, r, pl.ds(32 + g * 16, 16)]
        m_v[b, r, pl.ds(g * 16, 16)] = jnp.maximum(xlo + _lo(we_a), 0.0)
        m_v[b, r, pl.ds(32 + g * 16, 16)] = jnp.maximum(xhi + _lo(we_b), 0.0)
        m_v[b, r, pl.ds(64 + g * 16, 16)] = jnp.maximum(xlo + _hi(we_a), 0.0)
        m_v[b, r, pl.ds(96 + g * 16, 16)] = jnp.maximum(xhi + _hi(we_b), 0.0)


def _node1_body(x_ref, p_ref, wa_ref, ba_ref, wb_ref, bb_ref, h_ref, hw_ref):
    h1 = x_ref[...] + p_ref[0] + p_ref[1]
    t = jnp.maximum(jnp.dot(h1, wa_ref[...], preferred_element_type=jnp.float32)
                    + ba_ref[...], 0.0)
    g = jnp.dot(t, wb_ref[...], preferred_element_type=jnp.float32) + bb_ref[...]
    h = jnp.maximum(g, 0.0)
    h_ref[...] = h
    hw_ref[...] = _pack_tc(h)


def _node1(x, P1, W1a, b1a, W1b, b1b, nb):
    n, d = x.shape
    hdim = W1a.shape[1]
    grid = n // nb
    return pl.pallas_call(
        _node1_body,
        grid=(grid,),
        in_specs=[
            pl.BlockSpec((nb, d), lambda i: (i, 0)),
            pl.BlockSpec((NC, nb, d), lambda i: (0, i, 0)),
            pl.BlockSpec(W1a.shape, lambda i: (0, 0)),
            pl.BlockSpec(b1a.shape, lambda i: (0, 0)),
            pl.BlockSpec(W1b.shape, lambda i: (0, 0)),
            pl.BlockSpec(b1b.shape, lambda i: (0, 0)),
        ],
        out_specs=[
            pl.BlockSpec((nb, hdim), lambda i: (i, 0)),
            pl.BlockSpec((nb, hdim // 2), lambda i: (i, 0)),
        ],
        out_shape=[
            jax.ShapeDtypeStruct((n, hdim), jnp.float32),
            jax.ShapeDtypeStruct((n, hdim // 2), jnp.int32),
        ],
    )(x, P1, W1a, b1a, W1b, b1b)


def _node2_body(h_ref, p_ref, wm1_ref, bm1_ref, wm2_ref, bm2_ref,
                wl1_ref, bl1_ref, wl2_ref, bl2_ref, mu_ref, ls_ref):
    hb = h_ref[...]
    hm = hb + p_ref[0, :, :64] + p_ref[1, :, :64]
    hl = hb + p_ref[0, :, 64:] + p_ref[1, :, 64:]
    tm = jnp.maximum(jnp.dot(hm, wm1_ref[...], preferred_element_type=jnp.float32)
                     + bm1_ref[...], 0.0)
    mu_ref[...] = jnp.dot(tm, wm2_ref[...], preferred_element_type=jnp.float32) + bm2_ref[...]
    tl = jnp.maximum(jnp.dot(hl, wl1_ref[...], preferred_element_type=jnp.float32)
                     + bl1_ref[...], 0.0)
    ls = jnp.dot(tl, wl2_ref[...], preferred_element_type=jnp.float32) + bl2_ref[...]
    ls_ref[...] = jnp.clip(ls, -10.0, 10.0)


def _node2(h, P2, Wm1, bm1, Wm2, bm2, Wl1, bl1, Wl2, bl2, nb):
    n, hdim = h.shape
    ldim = Wm2.shape[1]
    grid = n // nb
    wspec = lambda shp: pl.BlockSpec(shp, lambda i: (0, 0))
    return pl.pallas_call(
        _node2_body,
        grid=(grid,),
        in_specs=[
            pl.BlockSpec((nb, hdim), lambda i: (i, 0)),
            pl.BlockSpec((NC, nb, 128), lambda i: (0, i, 0)),
            wspec(Wm1.shape), wspec(bm1.shape), wspec(Wm2.shape), wspec(bm2.shape),
            wspec(Wl1.shape), wspec(bl1.shape), wspec(Wl2.shape), wspec(bl2.shape),
        ],
        out_specs=[
            pl.BlockSpec((nb, ldim), lambda i: (i, 0)),
            pl.BlockSpec((nb, ldim), lambda i: (i, 0)),
        ],
        out_shape=[
            jax.ShapeDtypeStruct((n, ldim), jnp.float32),
            jax.ShapeDtypeStruct((n, ldim), jnp.float32),
        ],
    )(h, P2, Wm1, bm1, Wm2, bm2, Wl1, bl1, Wl2, bl2)


@jax.jit
def kernel(x, edge_index, edge_attr, We1, be1, W1a, b1a, W1b, b1b,
           Wem, bem, Wm1, bm1, Wm2, bm2, Wel, bel, Wl1, bl1, Wl2, bl2):
    n, d = x.shape
    e = edge_index.shape[1]
    ed = edge_attr.shape[1]

    ng = _ceil_to((e + NW * G - 1) // (NW * G), CH)
    e_pad = NW * ng * G
    n_acc = _ceil_to(n + 1, NS * 128)

    pad = e_pad - e
    src = edge_index[0].astype(jnp.int32)
    dst = edge_index[1].astype(jnp.int32)
    src2 = jnp.concatenate([src, jnp.zeros((pad,), jnp.int32)]).reshape(e_pad // G, G)
    dst2 = jnp.concatenate([dst, jnp.full((pad,), n, jnp.int32)]).reshape(e_pad // G, G)
    ea_p = jnp.concatenate([edge_attr, jnp.zeros((pad, ed), jnp.float32)])

    Wml = jnp.concatenate([Wem, Wel], axis=1)
    bml = jnp.concatenate([bem, bel]).reshape(1, 128)

    e1w = _edense(ea_p, We1, be1.reshape(1, -1), eb=8192)
    xw = _xpack(x, nb=2000)

    P1 = _make_agg(64, 64, _rows_l1, n_acc, ng)(xw, src2, dst2, e1w)
    emlw = _edense(ea_p, Wml, bml, eb=8192)
    h, hw = _node1(x, P1, W1a, b1a.reshape(1, -1),
                   W1b, b1b.reshape(1, -1), nb=2000)

    P2 = _make_agg(32, 64, _rows_l23, n_acc, ng)(hw, src2, dst2, emlw)
    mu, logstd = _node2(h, P2, Wm1, bm1.reshape(1, -1),
                        Wm2, bm2.reshape(1, -1), Wl1, bl1.reshape(1, -1),
                        Wl2, bl2.reshape(1, -1), nb=2000)
    return (mu, logstd)


# speedup vs baseline: 1.0083x; 1.0083x over previous
import functools
import jax
import jax.numpy as jnp
from jax import lax
from jax.experimental import pallas as pl
from jax.experimental.pallas import tpu as pltpu
from jax.experimental.pallas import tpu_sc as plsc

NC = 2
NS = 16
NW = NC * NS
G = 64
CH = 16

_MASK_HI = -65536


def _ceil_to(a, b):
    return (a + b - 1) // b * b


def _lo(w):
    return plsc.bitcast(w << 16, jnp.float32)


def _hi(w):
    return plsc.bitcast(w & _MASK_HI, jnp.float32)


def _pack_tc(v):
    u = lax.bitcast_convert_type(v.astype(jnp.bfloat16), jnp.uint16)
    k = u.shape[-1] // 2
    return (u[:, :k].astype(jnp.int32) |
            (u[:, k:].astype(jnp.int32) << 16))


def _edense_body(ea_ref, w_ref, b_ref, e_ref):
    v = jnp.dot(ea_ref[...], w_ref[...], preferred_element_type=jnp.float32)
    v = v + b_ref[...]
    e_ref[...] = _pack_tc(v)


def _edense(ea, Wc, bc, e_pad):
    e = ea.shape[0]
    eb = next(d for d in range(4096, 7, -8) if e % d == 0)
    grid = e // eb
    return pl.pallas_call(
        _edense_body,
        grid=(grid,),
        in_specs=[
            pl.BlockSpec((eb, ea.shape[1]), lambda i: (i, 0)),
            pl.BlockSpec(Wc.shape, lambda i: (0, 0)),
            pl.BlockSpec(bc.shape, lambda i: (0, 0)),
        ],
        out_specs=pl.BlockSpec((eb, 64), lambda i: (i, 0)),
        out_shape=jax.ShapeDtypeStruct((e_pad, 64), jnp.int32),
    )(ea, Wc, bc)


def _xpack_body(x_ref, xw_ref):
    xw_ref[...] = _pack_tc(x_ref[...])


def _xpack(x, nb):
    n, d = x.shape
    return pl.pallas_call(
        _xpack_body,
        grid=(n // nb,),
        in_specs=[pl.BlockSpec((nb, d), lambda i: (i, 0))],
        out_specs=pl.BlockSpec((nb, d // 2), lambda i: (i, 0)),
        out_shape=jax.ShapeDtypeStruct((n, d // 2), jnp.int32),
    )(x)


def _make_agg(tw, ew, compute_rows, n_acc, ng):
    mesh = plsc.VectorSubcoreMesh(core_axis_name="c", subcore_axis_name="s")
    rows_per_tile = n_acc // NS
    nz = rows_per_tile // 128
    nsteps = ng
    assert nsteps % 2 == 0 and CH % 2 == 0 and nsteps % CH == 0

    @functools.partial(
        pl.kernel,
        mesh=mesh,
        out_type=jax.ShapeDtypeStruct((NC, n_acc, 128), jnp.float32),
        compiler_params=pltpu.CompilerParams(use_tc_tiling_on_sc=False,
                                             needs_layout_passes=False),
        scratch_types=[
            pltpu.VMEM((CH, G), jnp.int32),
            pltpu.VMEM((2, CH, G), jnp.int32),
            pltpu.VMEM((2, G, tw), jnp.int32),
            pltpu.VMEM((2, G, ew), jnp.int32),
            pltpu.VMEM((2, G, 128), jnp.float32),
            pltpu.SemaphoreType.DMA,
            pltpu.SemaphoreType.DMA,
            pltpu.SemaphoreType.DMA,
            pltpu.SemaphoreType.DMA,
            pltpu.SemaphoreType.DMA,
            pltpu.SemaphoreType.DMA,
            pltpu.VMEM_SHARED((n_acc, 128), jnp.float32),
        ],
    )
    def body(tbl_hbm, src_hbm, dst_hbm, emb_hbm, out_hbm,
             src_v, dst_v, tbl_v, emb_v, m_v, gsem0, gsem1, esem0, esem1,
             ssem0, ssem1, acc):
        c = lax.axis_index("c")
        s = lax.axis_index("s")
        w = s * NC + c
        gsem = (gsem0, gsem1)
        esem = (esem0, esem1)
        ssem = (ssem0, ssem1)

        zeros16 = jnp.zeros((16,), jnp.float32)

        def zb_body(r, carry):
            for cc in range(8):
                m_v[0, r, pl.ds(cc * 16, 16)] = zeros16
            return carry

        lax.fori_loop(0, G, zb_body, 0)
        tile_base = s * rows_per_tile

        def zacc_body(k, carry):
            for q in range(128 // G):
                pltpu.sync_copy(
                    m_v.at[0],
                    acc.at[pl.ds(tile_base + k * 128 + q * G, G)])
            return carry

        lax.fori_loop(0, nz, zacc_body, 0)
        plsc.subcore_barrier()

        def issue_loads(j, b):
            jj = lax.rem(j, CH)
            pltpu.async_copy(
                emb_hbm.at[pl.ds((w * ng + j) * G, G)], emb_v.at[b], esem[b])
            pltpu.async_copy(tbl_hbm.at[src_v.at[jj]], tbl_v.at[b], gsem[b])

        def wait_loads(b):
            pltpu.make_async_copy(
                emb_hbm.at[pl.ds(0, G)], emb_v.at[b], esem[b]).wait()
            pltpu.make_async_copy(
                tbl_hbm.at[src_v.at[0]], tbl_v.at[b], gsem[b]).wait()

        def issue_scatter(j, b):
            jj = lax.rem(j, CH)
            p = lax.rem(j // CH, 2)
            pltpu.async_copy(m_v.at[b], acc.at[dst_v.at[p, jj]], ssem[b],
                             add=True)

        def wait_scatter(b):
            pltpu.make_async_copy(
                m_v.at[b], acc.at[dst_v.at[0, 0]], ssem[b]).wait()

        def load_idx(blk):
            pltpu.sync_copy(src_hbm.at[pl.ds(w * ng + blk * CH, CH)], src_v)
            pltpu.sync_copy(dst_hbm.at[pl.ds(w * ng + blk * CH, CH)],
                            dst_v.at[lax.rem(blk, 2)])

        load_idx(0)
        issue_loads(0, 0)

        def step2(jj, carry):
            for b in range(2):
                j = jj * 2 + b
                wait_loads(b)
                if b == 0:
                    issue_loads(j + 1, 1)
                else:
                    nxt = j + 1

                    @pl.when(nxt < nsteps)
                    def _():
                        @pl.when(lax.rem(nxt, CH) == 0)
                        def _():
                            load_idx(nxt // CH)

                        issue_loads(nxt, 0)

                @pl.when(j >= 2)
                def _(b=b):
                    wait_scatter(b)

                def rows(r, rc, b=b):
                    compute_rows(tbl_v, emb_v, m_v, b, r)
                    return rc

                lax.fori_loop(0, G, rows, 0)
                issue_scatter(j, b)
            return carry

        lax.fori_loop(0, nsteps // 2, step2, 0)
        wait_scatter(0)
        wait_scatter(1)
        plsc.subcore_barrier()

        def wout(k, carry):
            r0 = tile_base + k * 128
            pltpu.sync_copy(acc.at[pl.ds(r0, 128)],
                            out_hbm.at[c, pl.ds(r0, 128)])
            return carry

        lax.fori_loop(0, nz, wout, 0)

    return body


def _rows_l1(tbl_v, emb_v, m_v, b, r):
    for g in range(4):
        wx = tbl_v[b, r, pl.ds(g * 16, 16)]
        we = emb_v[b, r, pl.ds(g * 16, 16)]
        m_v[b, r, pl.ds(g * 16, 16)] = jnp.maximum(_lo(wx) + _lo(we), 0.0)
        m_v[b, r, pl.ds(64 + g * 16, 16)] = jnp.maximum(_hi(wx) + _hi(we), 0.0)


def _rows_l23(tbl_v, emb_v, m_v, b, r):
    for g in range(2):
        wx = tbl_v[b, r, pl.ds(g * 16, 16)]
        xlo = _lo(wx)
        xhi = _hi(wx)
        we_a = emb_v[b, r, pl.ds(g * 16, 16)]
        we_b = emb_v[b, r, pl.ds(32 + g * 16, 16)]
        m_v[b, r, pl.ds(g * 16, 16)] = jnp.maximum(xlo + _lo(we_a), 0.0)
        m_v[b, r, pl.ds(32 + g * 16, 16)] = jnp.maximum(xhi + _lo(we_b), 0.0)
        m_v[b, r, pl.ds(64 + g * 16, 16)] = jnp.maximum(xlo + _hi(we_a), 0.0)
        m_v[b, r, pl.ds(96 + g * 16, 16)] = jnp.maximum(xhi + _hi(we_b), 0.0)


def _node1_body(x_ref, p_ref, wa_ref, ba_ref, wb_ref, bb_ref, h_ref, hw_ref):
    h1 = x_ref[...] + p_ref[0] + p_ref[1]
    t = jnp.maximum(jnp.dot(h1, wa_ref[...], preferred_element_type=jnp.float32)
                    + ba_ref[...], 0.0)
    g = jnp.dot(t, wb_ref[...], preferred_element_type=jnp.float32) + bb_ref[...]
    h = jnp.maximum(g, 0.0)
    h_ref[...] = h
    hw_ref[...] = _pack_tc(h)


def _node1(x, P1, W1a, b1a, W1b, b1b, nb):
    n, d = x.shape
    hdim = W1a.shape[1]
    grid = n // nb
    return pl.pallas_call(
        _node1_body,
        grid=(grid,),
        in_specs=[
            pl.BlockSpec((nb, d), lambda i: (i, 0)),
            pl.BlockSpec((NC, nb, d), lambda i: (0, i, 0)),
            pl.BlockSpec(W1a.shape, lambda i: (0, 0)),
            pl.BlockSpec(b1a.shape, lambda i: (0, 0)),
            pl.BlockSpec(W1b.shape, lambda i: (0, 0)),
            pl.BlockSpec(b1b.shape, lambda i: (0, 0)),
        ],
        out_specs=[
            pl.BlockSpec((nb, hdim), lambda i: (i, 0)),
            pl.BlockSpec((nb, hdim // 2), lambda i: (i, 0)),
        ],
        out_shape=[
            jax.ShapeDtypeStruct((n, hdim), jnp.float32),
            jax.ShapeDtypeStruct((n, hdim // 2), jnp.int32),
        ],
    )(x, P1, W1a, b1a, W1b, b1b)


def _node2_body(h_ref, p_ref, wm1_ref, bm1_ref, wm2_ref, bm2_ref,
                wl1_ref, bl1_ref, wl2_ref, bl2_ref, mu_ref, ls_ref):
    hb = h_ref[...]
    hm = hb + p_ref[0, :, :64] + p_ref[1, :, :64]
    hl = hb + p_ref[0, :, 64:] + p_ref[1, :, 64:]
    tm = jnp.maximum(jnp.dot(hm, wm1_ref[...], preferred_element_type=jnp.float32)
                     + bm1_ref[...], 0.0)
    mu_ref[...] = jnp.dot(tm, wm2_ref[...], preferred_element_type=jnp.float32) + bm2_ref[...]
    tl = jnp.maximum(jnp.dot(hl, wl1_ref[...], preferred_element_type=jnp.float32)
                     + bl1_ref[...], 0.0)
    ls = jnp.dot(tl, wl2_ref[...], preferred_element_type=jnp.float32) + bl2_ref[...]
    ls_ref[...] = jnp.clip(ls, -10.0, 10.0)


def _node2(h, P2, Wm1, bm1, Wm2, bm2, Wl1, bl1, Wl2, bl2, nb):
    n, hdim = h.shape
    ldim = Wm2.shape[1]
    grid = n // nb
    wspec = lambda shp: pl.BlockSpec(shp, lambda i: (0, 0))
    return pl.pallas_call(
        _node2_body,
        grid=(grid,),
        in_specs=[
            pl.BlockSpec((nb, hdim), lambda i: (i, 0)),
            pl.BlockSpec((NC, nb, 128), lambda i: (0, i, 0)),
            wspec(Wm1.shape), wspec(bm1.shape), wspec(Wm2.shape), wspec(bm2.shape),
            wspec(Wl1.shape), wspec(bl1.shape), wspec(Wl2.shape), wspec(bl2.shape),
        ],
        out_specs=[
            pl.BlockSpec((nb, ldim), lambda i: (i, 0)),
            pl.BlockSpec((nb, ldim), lambda i: (i, 0)),
        ],
        out_shape=[
            jax.ShapeDtypeStruct((n, ldim), jnp.float32),
            jax.ShapeDtypeStruct((n, ldim), jnp.float32),
        ],
    )(h, P2, Wm1, bm1, Wm2, bm2, Wl1, bl1, Wl2, bl2)


@jax.jit
def kernel(x, edge_index, edge_attr, We1, be1, W1a, b1a, W1b, b1b,
           Wem, bem, Wm1, bm1, Wm2, bm2, Wel, bel, Wl1, bl1, Wl2, bl2):
    n, d = x.shape
    e = edge_index.shape[1]
    ed = edge_attr.shape[1]

    ng = _ceil_to((e + NW * G - 1) // (NW * G), CH)
    e_pad = NW * ng * G
    n_acc = _ceil_to(n + 1, NS * 128)

    pad = e_pad - e
    src = edge_index[0].astype(jnp.int32)
    dst = edge_index[1].astype(jnp.int32)
    src2 = jnp.concatenate([src, jnp.zeros((pad,), jnp.int32)]).reshape(e_pad // G, G)
    dst2 = jnp.concatenate([dst, jnp.full((pad,), n, jnp.int32)]).reshape(e_pad // G, G)

    Wml = jnp.concatenate([Wem, Wel], axis=1)
    bml = jnp.concatenate([bem, bel]).reshape(1, 128)

    e1w = _edense(edge_attr, We1, be1.reshape(1, -1), e_pad)
    xw = _xpack(x, nb=2000)

    P1 = _make_agg(64, 64, _rows_l1, n_acc, ng)(xw, src2, dst2, e1w)
    emlw = _edense(edge_attr, Wml, bml, e_pad)
    h, hw = _node1(x, P1, W1a, b1a.reshape(1, -1),
                   W1b, b1b.reshape(1, -1), nb=2000)

    P2 = _make_agg(32, 64, _rows_l23, n_acc, ng)(hw, src2, dst2, emlw)
    mu, logstd = _node2(h, P2, Wm1, bm1.reshape(1, -1),
                        Wm2, bm2.reshape(1, -1), Wl1, bl1.reshape(1, -1),
                        Wl2, bl2.reshape(1, -1), nb=2000)
    return (mu, logstd)


# speedup vs baseline: 1.3444x; 1.3333x over previous
import functools
import jax
import jax.numpy as jnp
from jax import lax
from jax.experimental import pallas as pl
from jax.experimental.pallas import tpu as pltpu
from jax.experimental.pallas import tpu_sc as plsc

NC = 2
NS = 16
NW = NC * NS
G = 64
CH = 16

_MASK_HI = -65536


def _ceil_to(a, b):
    return (a + b - 1) // b * b


def _lo(w):
    return plsc.bitcast(w << 16, jnp.float32)


def _hi(w):
    return plsc.bitcast(w & _MASK_HI, jnp.float32)


def _pack_tc(v):
    u = lax.bitcast_convert_type(v.astype(jnp.bfloat16), jnp.uint16)
    k = u.shape[-1] // 2
    return (u[:, :k].astype(jnp.int32) |
            (u[:, k:].astype(jnp.int32) << 16))


def _edense_body(ea_a_ref, ea_b_ref, w_ref, b_ref, e_ref):
    va = jnp.dot(ea_a_ref[...], w_ref[...], preferred_element_type=jnp.float32)
    vb = jnp.dot(ea_b_ref[...], w_ref[...], preferred_element_type=jnp.float32)
    va = va + b_ref[...]
    vb = vb + b_ref[...]
    e_ref[...] = jnp.concatenate([_pack_tc(va), _pack_tc(vb)], axis=1)


def _edense(ea, Wc, bc, e_pad):
    e = ea.shape[0]
    half = e_pad // 2
    eb = 2048
    grid = half // eb
    bmax = (e + eb - 1) // eb - 1

    return pl.pallas_call(
        _edense_body,
        grid=(grid,),
        in_specs=[
            pl.BlockSpec((eb, ea.shape[1]), lambda i: (i, 0)),
            pl.BlockSpec((eb, ea.shape[1]),
                         lambda i: (jnp.minimum(i + grid, bmax), 0)),
            pl.BlockSpec(Wc.shape, lambda i: (0, 0)),
            pl.BlockSpec(bc.shape, lambda i: (0, 0)),
        ],
        out_specs=pl.BlockSpec((eb, 128), lambda i: (i, 0)),
        out_shape=jax.ShapeDtypeStruct((half, 128), jnp.int32),
    )(ea, ea, Wc, bc)


def _xpack_body(x_ref, xw_ref):
    xw_ref[...] = _pack_tc(x_ref[...])


def _xpack(x, nb):
    n, d = x.shape
    return pl.pallas_call(
        _xpack_body,
        grid=(n // nb,),
        in_specs=[pl.BlockSpec((nb, d), lambda i: (i, 0))],
        out_specs=pl.BlockSpec((nb, d // 2), lambda i: (i, 0)),
        out_shape=jax.ShapeDtypeStruct((n, d // 2), jnp.int32),
    )(x)


def _make_agg(tw, ew, compute_rows, n_acc, ng):
    mesh = plsc.VectorSubcoreMesh(core_axis_name="c", subcore_axis_name="s")
    rows_per_tile = n_acc // NS
    nz = rows_per_tile // 128
    nsteps = ng
    assert nsteps % 2 == 0 and CH % 2 == 0 and nsteps % CH == 0

    @functools.partial(
        pl.kernel,
        mesh=mesh,
        out_type=jax.ShapeDtypeStruct((NC, n_acc, 128), jnp.float32),
        compiler_params=pltpu.CompilerParams(use_tc_tiling_on_sc=False,
                                             needs_layout_passes=False),
        scratch_types=[
            pltpu.VMEM((CH, G), jnp.int32),
            pltpu.VMEM((2, CH, G), jnp.int32),
            pltpu.VMEM((2, G, tw), jnp.int32),
            pltpu.VMEM((2, G // 2, 128), jnp.int32),
            pltpu.VMEM((2, G, 128), jnp.float32),
            pltpu.SemaphoreType.DMA,
            pltpu.SemaphoreType.DMA,
            pltpu.SemaphoreType.DMA,
            pltpu.SemaphoreType.DMA,
            pltpu.SemaphoreType.DMA,
            pltpu.SemaphoreType.DMA,
            pltpu.VMEM_SHARED((n_acc, 128), jnp.float32),
        ],
    )
    def body(tbl_hbm, src_hbm, dst_hbm, emb_hbm, out_hbm,
             src_v, dst_v, tbl_v, emb_v, m_v, gsem0, gsem1, esem0, esem1,
             ssem0, ssem1, acc):
        c = lax.axis_index("c")
        s = lax.axis_index("s")
        w = s * NC + c
        gsem = (gsem0, gsem1)
        esem = (esem0, esem1)
        ssem = (ssem0, ssem1)

        zeros16 = jnp.zeros((16,), jnp.float32)

        def zb_body(r, carry):
            for cc in range(8):
                m_v[0, r, pl.ds(cc * 16, 16)] = zeros16
            return carry

        lax.fori_loop(0, G, zb_body, 0)
        tile_base = s * rows_per_tile

        def zacc_body(k, carry):
            for q in range(128 // G):
                pltpu.sync_copy(
                    m_v.at[0],
                    acc.at[pl.ds(tile_base + k * 128 + q * G, G)])
            return carry

        lax.fori_loop(0, nz, zacc_body, 0)
        plsc.subcore_barrier()

        def issue_loads(j, b):
            jj = lax.rem(j, CH)
            pltpu.async_copy(
                emb_hbm.at[pl.ds((w * ng + j) * (G // 2), G // 2)],
                emb_v.at[b], esem[b])
            pltpu.async_copy(tbl_hbm.at[src_v.at[jj]], tbl_v.at[b], gsem[b])

        def wait_loads(b):
            pltpu.make_async_copy(
                emb_hbm.at[pl.ds(0, G // 2)], emb_v.at[b], esem[b]).wait()
            pltpu.make_async_copy(
                tbl_hbm.at[src_v.at[0]], tbl_v.at[b], gsem[b]).wait()

        def issue_scatter(j, b):
            jj = lax.rem(j, CH)
            p = lax.rem(j // CH, 2)
            pltpu.async_copy(m_v.at[b], acc.at[dst_v.at[p, jj]], ssem[b],
                             add=True)

        def wait_scatter(b):
            pltpu.make_async_copy(
                m_v.at[b], acc.at[dst_v.at[0, 0]], ssem[b]).wait()

        def load_idx(blk):
            pltpu.sync_copy(src_hbm.at[pl.ds(w * ng + blk * CH, CH)], src_v)
            pltpu.sync_copy(dst_hbm.at[pl.ds(w * ng + blk * CH, CH)],
                            dst_v.at[lax.rem(blk, 2)])

        load_idx(0)
        issue_loads(0, 0)

        def step2(jj, carry):
            for b in range(2):
                j = jj * 2 + b
                wait_loads(b)
                if b == 0:
                    issue_loads(j + 1, 1)
                else:
                    nxt = j + 1

                    @pl.when(nxt < nsteps)
                    def _():
                        @pl.when(lax.rem(nxt, CH) == 0)
                        def _():
                            load_idx(nxt // CH)

                        issue_loads(nxt, 0)

                @pl.when(j >= 2)
                def _(b=b):
                    wait_scatter(b)

                def rows(r, rc, b=b):
                    compute_rows(tbl_v, emb_v, m_v, b, r)
                    return rc

                lax.fori_loop(0, G // 2, rows, 0)
                issue_scatter(j, b)
            return carry

        lax.fori_loop(0, nsteps // 2, step2, 0)
        wait_scatter(0)
        wait_scatter(1)
        plsc.subcore_barrier()

        def wout(k, carry):
            r0 = tile_base + k * 128
            pltpu.sync_copy(acc.at[pl.ds(r0, 128)],
                            out_hbm.at[c, pl.ds(r0, 128)])
            return carry

        lax.fori_loop(0, nz, wout, 0)

    return body


def _rows_l1(tbl_v, emb_v, m_v, b, r):
    for (ro, wo) in ((0, 0), (G // 2, 64)):
        for g in range(4):
            wx = tbl_v[b, ro + r, pl.ds(g * 16, 16)]
            we = emb_v[b, r, pl.ds(wo + g * 16, 16)]
            m_v[b, ro + r, pl.ds(g * 16, 16)] = jnp.maximum(
                _lo(wx) + _lo(we), 0.0)
            m_v[b, ro + r, pl.ds(64 + g * 16, 16)] = jnp.maximum(
                _hi(wx) + _hi(we), 0.0)


def _rows_l23(tbl_v, emb_v, m_v, b, r):
    for (ro, wo) in ((0, 0), (G // 2, 64)):
        for g in range(2):
            wx = tbl_v[b, ro + r, pl.ds(g * 16, 16)]
            xlo = _lo(wx)
            xhi = _hi(wx)
            we_a = emb_v[b, r, pl.ds(wo + g * 16, 16)]
            we_b = emb_v[b, r, pl.ds(wo + 32 + g * 16, 16)]
            m_v[b, ro + r, pl.ds(g * 16, 16)] = jnp.maximum(xlo + _lo(we_a), 0.0)
            m_v[b, ro + r, pl.ds(32 + g * 16, 16)] = jnp.maximum(xhi + _lo(we_b), 0.0)
            m_v[b, ro + r, pl.ds(64 + g * 16, 16)] = jnp.maximum(xlo + _hi(we_a), 0.0)
            m_v[b, ro + r, pl.ds(96 + g * 16, 16)] = jnp.maximum(xhi + _hi(we_b), 0.0)


def _node1_body(x_ref, p_ref, wa_ref, ba_ref, wb_ref, bb_ref, h_ref, hw_ref):
    h1 = x_ref[...] + p_ref[0] + p_ref[1]
    t = jnp.maximum(jnp.dot(h1, wa_ref[...], preferred_element_type=jnp.float32)
                    + ba_ref[...], 0.0)
    g = jnp.dot(t, wb_ref[...], preferred_element_type=jnp.float32) + bb_ref[...]
    h = jnp.maximum(g, 0.0)
    h_ref[...] = h
    hw_ref[...] = _pack_tc(h)


def _node1(x, P1, W1a, b1a, W1b, b1b, nb):
    n, d = x.shape
    hdim = W1a.shape[1]
    grid = n // nb
    return pl.pallas_call(
        _node1_body,
        grid=(grid,),
        in_specs=[
            pl.BlockSpec((nb, d), lambda i: (i, 0)),
            pl.BlockSpec((NC, nb, d), lambda i: (0, i, 0)),
            pl.BlockSpec(W1a.shape, lambda i: (0, 0)),
            pl.BlockSpec(b1a.shape, lambda i: (0, 0)),
            pl.BlockSpec(W1b.shape, lambda i: (0, 0)),
            pl.BlockSpec(b1b.shape, lambda i: (0, 0)),
        ],
        out_specs=[
            pl.BlockSpec((nb, hdim), lambda i: (i, 0)),
            pl.BlockSpec((nb, hdim // 2), lambda i: (i, 0)),
        ],
        out_shape=[
            jax.ShapeDtypeStruct((n, hdim), jnp.float32),
            jax.ShapeDtypeStruct((n, hdim // 2), jnp.int32),
        ],
    )(x, P1, W1a, b1a, W1b, b1b)


def _node2_body(h_ref, p_ref, wm1_ref, bm1_ref, wm2_ref, bm2_ref,
                wl1_ref, bl1_ref, wl2_ref, bl2_ref, mu_ref, ls_ref):
    hb = h_ref[...]
    hm = hb + p_ref[0, :, :64] + p_ref[1, :, :64]
    hl = hb + p_ref[0, :, 64:] + p_ref[1, :, 64:]
    tm = jnp.maximum(jnp.dot(hm, wm1_ref[...], preferred_element_type=jnp.float32)
                     + bm1_ref[...], 0.0)
    mu_ref[...] = jnp.dot(tm, wm2_ref[...], preferred_element_type=jnp.float32) + bm2_ref[...]
    tl = jnp.maximum(jnp.dot(hl, wl1_ref[...], preferred_element_type=jnp.float32)
                     + bl1_ref[...], 0.0)
    ls = jnp.dot(tl, wl2_ref[...], preferred_element_type=jnp.float32) + bl2_ref[...]
    ls_ref[...] = jnp.clip(ls, -10.0, 10.0)


def _node2(h, P2, Wm1, bm1, Wm2, bm2, Wl1, bl1, Wl2, bl2, nb):
    n, hdim = h.shape
    ldim = Wm2.shape[1]
    grid = n // nb
    wspec = lambda shp: pl.BlockSpec(shp, lambda i: (0, 0))
    return pl.pallas_call(
        _node2_body,
        grid=(grid,),
        in_specs=[
            pl.BlockSpec((nb, hdim), lambda i: (i, 0)),
            pl.BlockSpec((NC, nb, 128), lambda i: (0, i, 0)),
            wspec(Wm1.shape), wspec(bm1.shape), wspec(Wm2.shape), wspec(bm2.shape),
            wspec(Wl1.shape), wspec(bl1.shape), wspec(Wl2.shape), wspec(bl2.shape),
        ],
        out_specs=[
            pl.BlockSpec((nb, ldim), lambda i: (i, 0)),
            pl.BlockSpec((nb, ldim), lambda i: (i, 0)),
        ],
        out_shape=[
            jax.ShapeDtypeStruct((n, ldim), jnp.float32),
            jax.ShapeDtypeStruct((n, ldim), jnp.float32),
        ],
    )(h, P2, Wm1, bm1, Wm2, bm2, Wl1, bl1, Wl2, bl2)


@jax.jit
def kernel(x, edge_index, edge_attr, We1, be1, W1a, b1a, W1b, b1b,
           Wem, bem, Wm1, bm1, Wm2, bm2, Wel, bel, Wl1, bl1, Wl2, bl2):
    n, d = x.shape
    e = edge_index.shape[1]
    ed = edge_attr.shape[1]

    ng = _ceil_to((e + NW * G - 1) // (NW * G), CH)
    e_pad = NW * ng * G
    n_acc = _ceil_to(n + 1, NS * 128)

    pad = e_pad - e
    half = e_pad // 2
    src = edge_index[0].astype(jnp.int32)
    dst = edge_index[1].astype(jnp.int32)

    def _grp(v, fill):
        vp = jnp.concatenate([v, jnp.full((pad,), fill, jnp.int32)])
        return jnp.concatenate([vp[:half].reshape(-1, G // 2),
                                vp[half:].reshape(-1, G // 2)], axis=1)

    src2 = _grp(src, 0)
    dst2 = _grp(dst, n)

    Wml = jnp.concatenate([Wem, Wel], axis=1)
    bml = jnp.concatenate([bem, bel]).reshape(1, 128)

    e1w = _edense(edge_attr, We1, be1.reshape(1, -1), e_pad)
    xw = _xpack(x, nb=2000)

    P1 = _make_agg(64, 64, _rows_l1, n_acc, ng)(xw, src2, dst2, e1w)
    emlw = _edense(edge_attr, Wml, bml, e_pad)
    h, hw = _node1(x, P1, W1a, b1a.reshape(1, -1),
                   W1b, b1b.reshape(1, -1), nb=2000)

    P2 = _make_agg(32, 64, _rows_l23, n_acc, ng)(hw, src2, dst2, emlw)
    mu, logstd = _node2(h, P2, Wm1, bm1.reshape(1, -1),
                        Wm2, bm2.reshape(1, -1), Wl1, bl1.reshape(1, -1),
                        Wl2, bl2.reshape(1, -1), nb=2000)
    return (mu, logstd)
